# baseline (speedup 1.0000x reference)
import jax
import jax.numpy as jnp
from jax.experimental import pallas as pl

_OUT_CH = 64
_RPN_CH = 16
_A = 6
_ANCHOR_LENGTHS = (16.0, 32.0, 64.0, 128.0, 256.0, 512.0)
_STRIDE = 8
_PRE_TOPN = 6000
_NMS_THR = 0.7
_POST_TOPN = 300
_SEQ_LEN = 16384
_L = _SEQ_LEN // _STRIDE
_N = _L * _A
_ROWS = _N // 128
_NEG = -1e30


def _head_kernel(T_ref, W1_ref, bbb_ref, wd_ref, Wpw_ref, bpw_ref,
                 gam_ref, bet_ref, mu_ref, var_ref,
                 Wcls_ref, bcls_ref, Wb0_ref, bb0_ref, Wb1_ref, bb1_ref,
                 lens_ref, sc_ref, st_ref, en_ref):
    bf = jnp.bfloat16
    def mdot(a, b):
        return jnp.dot(a.astype(bf), b.astype(bf),
                       preferred_element_type=jnp.float32)
    feat = mdot(T_ref[...], W1_ref[...]) + bbb_ref[0:1, :]
    feat = jnp.maximum(feat, 0.0)
    featb = feat.astype(bf).astype(jnp.float32)
    wd = wd_ref[...]
    zrow = jnp.zeros((1, _OUT_CH), jnp.float32)
    fm1 = jnp.concatenate([zrow, featb[:-1, :]], axis=0)
    fp1 = jnp.concatenate([featb[1:, :], zrow], axis=0)
    x = fm1 * wd[0:1, :] + featb * wd[1:2, :] + fp1 * wd[2:3, :]
    x = mdot(x, Wpw_ref[...]) + bpw_ref[0:1, :]
    x = jnp.maximum(x, 0.0)
    x = gam_ref[0:1, :] * (x - mu_ref[0:1, :]) / jnp.sqrt(var_ref[0:1, :] + 1e-5) \
        + bet_ref[0:1, :]
    logits = mdot(x, Wcls_ref[...]) + bcls_ref[0:1, :]
    scores = jax.nn.sigmoid(logits)
    d0 = mdot(x, Wb0_ref[...]) + bb0_ref[0:1, :]
    d1 = mdot(x, Wb1_ref[...]) + bb1_ref[0:1, :]
    anc_c = (jax.lax.broadcasted_iota(jnp.int32, (_L, _A), 0).astype(jnp.float32)
             * _STRIDE + _STRIDE / 2.0)
    anc_l = lens_ref[0:1, :]
    new_c = anc_c + d0 * anc_l
    new_l = anc_l * jnp.exp(jnp.clip(d1, -10.0, 10.0))
    sc_ref[...] = scores
    st_ref[...] = jnp.clip(new_c - new_l / 2.0, 0.0, float(_SEQ_LEN))
    en_ref[...] = jnp.clip(new_c + new_l / 2.0, 0.0, float(_SEQ_LEN))


def _iv():
    return (jax.lax.broadcasted_iota(jnp.int32, (_ROWS, 128), 0) * 128
            + jax.lax.broadcasted_iota(jnp.int32, (_ROWS, 128), 1))


def _nms_kernel(sc_ref, st_ref, en_ref, out_ref):
    scores = sc_ref[...]
    ivals = _iv()
    sbits = jax.lax.bitcast_convert_type(scores, jnp.int32)

    def t_body(_, lh):
        lo, hi = lh
        mid = jax.lax.div(lo + hi, jnp.int32(2))
        cnt = jnp.sum(jnp.where(sbits >= mid, 1.0, 0.0))
        ge = cnt >= _PRE_TOPN
        return (jnp.where(ge, mid, lo), jnp.where(ge, hi, mid))

    tstar, _ = jax.lax.fori_loop(
        0, 31, t_body, (jnp.int32(0), jnp.int32(1 << 30)))
    cnt_gt = jnp.sum(jnp.where(sbits > tstar, 1.0, 0.0))
    need = jnp.float32(_PRE_TOPN) - cnt_gt
    tie = sbits == tstar

    def c_body(_, lh):
        lo, hi = lh
        mid = lo + jax.lax.div(hi - lo, jnp.int32(2))
        cnt = jnp.sum(jnp.where(tie & (ivals <= mid), 1.0, 0.0))
        ge = cnt >= need
        return (jnp.where(ge, lo, mid), jnp.where(ge, mid, hi))

    _, cstar = jax.lax.fori_loop(
        0, 15, c_body, (jnp.int32(-1), jnp.int32(_N - 1)))
    eligible = (sbits > tstar) | (tie & (ivals <= cstar))
    swork = jnp.where(eligible, scores, _NEG)

    li = jax.lax.broadcasted_iota(jnp.int32, (1, 128), 1)

    def _sel(a, b):
        swa, iva, sta, ena = a
        swb, ivb, stb, enb = b
        better = (swb > swa) | ((swb == swa) & (ivb < iva))
        return tuple(jnp.where(better, xb, xa)
                     for xa, xb in zip(a, b, strict=True))

    def _argmax4(state):
        for h in (48, 24):
            state = _sel(tuple(x[:h] for x in state),
                         tuple(x[h:2 * h] for x in state))
        state = _sel(_sel(tuple(x[0:8] for x in state),
                          tuple(x[8:16] for x in state)),
                     tuple(x[16:24] for x in state))
        w = 64
        while w >= 1:
            state = _sel(tuple(x[:, :w] for x in state),
                         tuple(x[:, w:2 * w] for x in state))
            w //= 2
        h = 4
        while h >= 1:
            state = _sel(tuple(x[:h] for x in state),
                         tuple(x[h:2 * h] for x in state))
            h //= 2
        return state

    def body(t, sw):
        iv = _iv()
        starts = st_ref[...]
        ends = en_ref[...]
        m, idx, st, en = _argmax4((sw, iv, starts, ends))
        inter = jnp.maximum(0.0, jnp.minimum(en, ends) - jnp.maximum(st, starts))
        union = (en - st) + (ends - starts) - inter
        iou = inter / jnp.maximum(union, 1e-8)
        sw = jnp.where((iou > _NMS_THR) | (iv == idx), _NEG, sw)
        row = jnp.where(li == 0, m, jnp.where(li == 1, st,
                        jnp.where(li == 2, en, 0.0)))
        prev = out_ref[0:1, :]
        out_ref[pl.ds(t, 1), :] = jnp.where(m > -1.0, row, prev)
        return sw

    jax.lax.fori_loop(0, _POST_TOPN, body, swork)


def _head_pallas(sequence, W_bb, b_bb, W_dw, W_pw, b_pw, bn_gamma, bn_beta,
                 bn_mean, bn_var, W_cls, b_cls, W_box, b_box):
    f32 = jnp.float32
    seq = sequence.reshape(_SEQ_LEN)
    xpad = jnp.pad(seq, (3, 4))
    taps = [jax.lax.slice(xpad, (t,), (t + 8 * (_L - 1) + 1,), (8,))
            for t in range(7)]
    T = jnp.stack(taps + [jnp.zeros((_L,), f32)], axis=1)
    W1 = jnp.concatenate([W_bb[:, 0, :].T, jnp.zeros((1, _OUT_CH), f32)], 0)
    wd = jnp.concatenate([W_dw[:, 0, :].T, jnp.zeros((5, _OUT_CH), f32)], 0)
    Wpw = W_pw[:, :, 0].T
    Wcls = W_cls[:, :, 0].T
    Wb0 = W_box[0::2, :, 0].T
    Wb1 = W_box[1::2, :, 0].T
    lens = jnp.asarray(_ANCHOR_LENGTHS, f32).reshape(1, _A)
    args = (T, W1, b_bb.reshape(1, -1), wd, Wpw, b_pw.reshape(1, -1),
            bn_gamma.reshape(1, -1), bn_beta.reshape(1, -1),
            bn_mean.reshape(1, -1), bn_var.reshape(1, -1),
            Wcls, b_cls.reshape(1, -1), Wb0, b_box[0::2].reshape(1, -1),
            Wb1, b_box[1::2].reshape(1, -1), lens)
    return pl.pallas_call(
        _head_kernel,
        out_shape=[jax.ShapeDtypeStruct((_L, _A), f32)] * 3,
    )(*args)


def _nms_pallas(s96, st96, en96):
    out = pl.pallas_call(
        _nms_kernel,
        out_shape=jax.ShapeDtypeStruct((_POST_TOPN + 4, 128), jnp.float32),
    )(s96, st96, en96)
    return out[:_POST_TOPN, :3][:, None, :]


def kernel(sequence, W_bb, b_bb, W_dw, W_pw, b_pw, bn_gamma, bn_beta,
           bn_mean, bn_var, W_cls, b_cls, W_box, b_box):
    sc, st, en = _head_pallas(sequence, W_bb, b_bb, W_dw, W_pw, b_pw,
                              bn_gamma, bn_beta, bn_mean, bn_var,
                              W_cls, b_cls, W_box, b_box)
    return _nms_pallas(sc.reshape(_ROWS, 128), st.reshape(_ROWS, 128),
                       en.reshape(_ROWS, 128))


# speedup vs baseline: 1.3122x; 1.3122x over previous
import jax
import jax.numpy as jnp
from jax.experimental import pallas as pl

_OUT_CH = 64
_RPN_CH = 16
_A = 6
_ANCHOR_LENGTHS = (16.0, 32.0, 64.0, 128.0, 256.0, 512.0)
_STRIDE = 8
_PRE_TOPN = 6000
_NMS_THR = 0.7
_POST_TOPN = 300
_SEQ_LEN = 16384
_L = _SEQ_LEN // _STRIDE
_N = _L * _A
_ROWS = _N // 128
_NEG = -1e30


def _head_kernel(T_ref, W1_ref, bbb_ref, wd_ref, Wpw_ref, bpw_ref,
                 gam_ref, bet_ref, mu_ref, var_ref,
                 Wcls_ref, bcls_ref, Wb0_ref, bb0_ref, Wb1_ref, bb1_ref,
                 lens_ref, sc_ref, st_ref, en_ref):
    bf = jnp.bfloat16
    def mdot(a, b):
        return jnp.dot(a.astype(bf), b.astype(bf),
                       preferred_element_type=jnp.float32)
    feat = mdot(T_ref[...], W1_ref[...]) + bbb_ref[0:1, :]
    feat = jnp.maximum(feat, 0.0)
    featb = feat.astype(bf).astype(jnp.float32)
    wd = wd_ref[...]
    zrow = jnp.zeros((1, _OUT_CH), jnp.float32)
    fm1 = jnp.concatenate([zrow, featb[:-1, :]], axis=0)
    fp1 = jnp.concatenate([featb[1:, :], zrow], axis=0)
    x = fm1 * wd[0:1, :] + featb * wd[1:2, :] + fp1 * wd[2:3, :]
    x = mdot(x, Wpw_ref[...]) + bpw_ref[0:1, :]
    x = jnp.maximum(x, 0.0)
    x = gam_ref[0:1, :] * (x - mu_ref[0:1, :]) / jnp.sqrt(var_ref[0:1, :] + 1e-5) \
        + bet_ref[0:1, :]
    logits = mdot(x, Wcls_ref[...]) + bcls_ref[0:1, :]
    scores = jax.nn.sigmoid(logits)
    d0 = mdot(x, Wb0_ref[...]) + bb0_ref[0:1, :]
    d1 = mdot(x, Wb1_ref[...]) + bb1_ref[0:1, :]
    anc_c = (jax.lax.broadcasted_iota(jnp.int32, (_L, _A), 0).astype(jnp.float32)
             * _STRIDE + _STRIDE / 2.0)
    anc_l = lens_ref[0:1, :]
    new_c = anc_c + d0 * anc_l
    new_l = anc_l * jnp.exp(jnp.clip(d1, -10.0, 10.0))
    sc_ref[...] = scores
    st_ref[...] = jnp.clip(new_c - new_l / 2.0, 0.0, float(_SEQ_LEN))
    en_ref[...] = jnp.clip(new_c + new_l / 2.0, 0.0, float(_SEQ_LEN))


def _iv():
    return (jax.lax.broadcasted_iota(jnp.int32, (_ROWS, 128), 0) * 128
            + jax.lax.broadcasted_iota(jnp.int32, (_ROWS, 128), 1))


def _nms_kernel(sc_ref, st_ref, en_ref, out_ref):
    scores = sc_ref[...]
    ivals = _iv()
    sbits = jax.lax.bitcast_convert_type(scores, jnp.int32)

    def t_body(_, lh):
        lo, hi = lh
        mid = jax.lax.div(lo + hi, jnp.int32(2))
        cnt = jnp.sum(jnp.where(sbits >= mid, 1.0, 0.0))
        ge = cnt >= _PRE_TOPN
        return (jnp.where(ge, mid, lo), jnp.where(ge, hi, mid))

    tstar, _ = jax.lax.fori_loop(
        0, 31, t_body, (jnp.int32(0), jnp.int32(1 << 30)))
    cnt_gt = jnp.sum(jnp.where(sbits > tstar, 1.0, 0.0))
    need = jnp.float32(_PRE_TOPN) - cnt_gt
    tie = sbits == tstar

    def c_body(_, lh):
        lo, hi = lh
        mid = lo + jax.lax.div(hi - lo, jnp.int32(2))
        cnt = jnp.sum(jnp.where(tie & (ivals <= mid), 1.0, 0.0))
        ge = cnt >= need
        return (jnp.where(ge, lo, mid), jnp.where(ge, mid, hi))

    _, cstar = jax.lax.fori_loop(
        0, 15, c_body, (jnp.int32(-1), jnp.int32(_N - 1)))
    eligible = (sbits > tstar) | (tie & (ivals <= cstar))
    swork = jnp.where(eligible, scores, _NEG)

    li = jax.lax.broadcasted_iota(jnp.int32, (1, 128), 1)

    def body(t, sw):
        m = jnp.max(sw)
        iv = _iv()
        idx = jnp.min(jnp.where(sw == m, iv, jnp.int32(1 << 30)))
        r = jax.lax.shift_right_logical(idx, 7)
        c = jax.lax.bitwise_and(idx, jnp.int32(127))
        sel_row = li == c
        sc = jnp.sum(jnp.where(sel_row, sc_ref[pl.ds(r, 1), :], 0.0))
        st = jnp.sum(jnp.where(sel_row, st_ref[pl.ds(r, 1), :], 0.0))
        en = jnp.sum(jnp.where(sel_row, en_ref[pl.ds(r, 1), :], 0.0))
        starts = st_ref[...]
        ends = en_ref[...]
        inter = jnp.maximum(0.0, jnp.minimum(en, ends) - jnp.maximum(st, starts))
        union = (en - st) + (ends - starts) - inter
        iou = inter / jnp.maximum(union, 1e-8)
        sw = jnp.where((iou > _NMS_THR) | (iv == idx), _NEG, sw)
        row = jnp.where(li == 0, sc, jnp.where(li == 1, st,
                        jnp.where(li == 2, en, 0.0)))
        prev = out_ref[0:1, :]
        out_ref[pl.ds(t, 1), :] = jnp.where(m > -1.0, row, prev)
        return sw

    jax.lax.fori_loop(0, _POST_TOPN, body, swork, unroll=2)


def _head_pallas(sequence, W_bb, b_bb, W_dw, W_pw, b_pw, bn_gamma, bn_beta,
                 bn_mean, bn_var, W_cls, b_cls, W_box, b_box):
    f32 = jnp.float32
    seq = sequence.reshape(_SEQ_LEN)
    xpad = jnp.pad(seq, (3, 4))
    taps = [jax.lax.slice(xpad, (t,), (t + 8 * (_L - 1) + 1,), (8,))
            for t in range(7)]
    T = jnp.stack(taps + [jnp.zeros((_L,), f32)], axis=1)
    W1 = jnp.concatenate([W_bb[:, 0, :].T, jnp.zeros((1, _OUT_CH), f32)], 0)
    wd = jnp.concatenate([W_dw[:, 0, :].T, jnp.zeros((5, _OUT_CH), f32)], 0)
    Wpw = W_pw[:, :, 0].T
    Wcls = W_cls[:, :, 0].T
    Wb0 = W_box[0::2, :, 0].T
    Wb1 = W_box[1::2, :, 0].T
    lens = jnp.asarray(_ANCHOR_LENGTHS, f32).reshape(1, _A)
    args = (T, W1, b_bb.reshape(1, -1), wd, Wpw, b_pw.reshape(1, -1),
            bn_gamma.reshape(1, -1), bn_beta.reshape(1, -1),
            bn_mean.reshape(1, -1), bn_var.reshape(1, -1),
            Wcls, b_cls.reshape(1, -1), Wb0, b_box[0::2].reshape(1, -1),
            Wb1, b_box[1::2].reshape(1, -1), lens)
    return pl.pallas_call(
        _head_kernel,
        out_shape=[jax.ShapeDtypeStruct((_L, _A), f32)] * 3,
    )(*args)


def _nms_pallas(s96, st96, en96):
    out = pl.pallas_call(
        _nms_kernel,
        out_shape=jax.ShapeDtypeStruct((_POST_TOPN + 4, 128), jnp.float32),
    )(s96, st96, en96)
    return out[:_POST_TOPN, :3][:, None, :]


def kernel(sequence, W_bb, b_bb, W_dw, W_pw, b_pw, bn_gamma, bn_beta,
           bn_mean, bn_var, W_cls, b_cls, W_box, b_box):
    sc, st, en = _head_pallas(sequence, W_bb, b_bb, W_dw, W_pw, b_pw,
                              bn_gamma, bn_beta, bn_mean, bn_var,
                              W_cls, b_cls, W_box, b_box)
    return _nms_pallas(sc.reshape(_ROWS, 128), st.reshape(_ROWS, 128),
                       en.reshape(_ROWS, 128))


# speedup vs baseline: 1.3136x; 1.0010x over previous
import jax
import jax.numpy as jnp
from jax.experimental import pallas as pl

_OUT_CH = 64
_RPN_CH = 16
_A = 6
_ANCHOR_LENGTHS = (16.0, 32.0, 64.0, 128.0, 256.0, 512.0)
_STRIDE = 8
_PRE_TOPN = 6000
_NMS_THR = 0.7
_POST_TOPN = 300
_SEQ_LEN = 16384
_L = _SEQ_LEN // _STRIDE
_N = _L * _A
_ROWS = _N // 128
_NEG = -1e30


def _head_kernel(T_ref, W1_ref, bbb_ref, wd_ref, Wpw_ref, bpw_ref,
                 gam_ref, bet_ref, mu_ref, var_ref,
                 Wcls_ref, bcls_ref, Wb0_ref, bb0_ref, Wb1_ref, bb1_ref,
                 lens_ref, sc_ref, st_ref, en_ref):
    bf = jnp.bfloat16
    def mdot(a, b):
        return jnp.dot(a.astype(bf), b.astype(bf),
                       preferred_element_type=jnp.float32)
    feat = mdot(T_ref[...], W1_ref[...]) + bbb_ref[0:1, :]
    feat = jnp.maximum(feat, 0.0)
    featb = feat.astype(bf).astype(jnp.float32)
    wd = wd_ref[...]
    zrow = jnp.zeros((1, _OUT_CH), jnp.float32)
    fm1 = jnp.concatenate([zrow, featb[:-1, :]], axis=0)
    fp1 = jnp.concatenate([featb[1:, :], zrow], axis=0)
    x = fm1 * wd[0:1, :] + featb * wd[1:2, :] + fp1 * wd[2:3, :]
    x = mdot(x, Wpw_ref[...]) + bpw_ref[0:1, :]
    x = jnp.maximum(x, 0.0)
    x = gam_ref[0:1, :] * (x - mu_ref[0:1, :]) / jnp.sqrt(var_ref[0:1, :] + 1e-5) \
        + bet_ref[0:1, :]
    logits = mdot(x, Wcls_ref[...]) + bcls_ref[0:1, :]
    scores = jax.nn.sigmoid(logits)
    d0 = mdot(x, Wb0_ref[...]) + bb0_ref[0:1, :]
    d1 = mdot(x, Wb1_ref[...]) + bb1_ref[0:1, :]
    anc_c = (jax.lax.broadcasted_iota(jnp.int32, (_L, _A), 0).astype(jnp.float32)
             * _STRIDE + _STRIDE / 2.0)
    anc_l = lens_ref[0:1, :]
    new_c = anc_c + d0 * anc_l
    new_l = anc_l * jnp.exp(jnp.clip(d1, -10.0, 10.0))
    sc_ref[...] = scores
    st_ref[...] = jnp.clip(new_c - new_l / 2.0, 0.0, float(_SEQ_LEN))
    en_ref[...] = jnp.clip(new_c + new_l / 2.0, 0.0, float(_SEQ_LEN))


def _iv():
    return (jax.lax.broadcasted_iota(jnp.int32, (_ROWS, 128), 0) * 128
            + jax.lax.broadcasted_iota(jnp.int32, (_ROWS, 128), 1))


def _nms_kernel(sc_ref, st_ref, en_ref, out_ref):
    scores = sc_ref[...]
    ivals = _iv()
    sbits = jax.lax.bitcast_convert_type(scores, jnp.int32)

    def t_body(_, lh):
        lo, hi = lh
        mid = jax.lax.div(lo + hi, jnp.int32(2))
        cnt = jnp.sum(jnp.where(sbits >= mid, 1.0, 0.0))
        ge = cnt >= _PRE_TOPN
        return (jnp.where(ge, mid, lo), jnp.where(ge, hi, mid))

    tstar, _ = jax.lax.fori_loop(
        0, 31, t_body, (jnp.int32(0), jnp.int32(1 << 30)))
    cnt_gt = jnp.sum(jnp.where(sbits > tstar, 1.0, 0.0))
    need = jnp.float32(_PRE_TOPN) - cnt_gt
    tie = sbits == tstar

    def c_body(_, lh):
        lo, hi = lh
        mid = lo + jax.lax.div(hi - lo, jnp.int32(2))
        cnt = jnp.sum(jnp.where(tie & (ivals <= mid), 1.0, 0.0))
        ge = cnt >= need
        return (jnp.where(ge, lo, mid), jnp.where(ge, mid, hi))

    _, cstar = jax.lax.fori_loop(
        0, 15, c_body, (jnp.int32(-1), jnp.int32(_N - 1)))
    eligible = (sbits > tstar) | (tie & (ivals <= cstar))
    swork = jnp.where(eligible, scores, _NEG)

    li = jax.lax.broadcasted_iota(jnp.int32, (1, 128), 1)

    def body(t, carry):
        sw, first_row = carry
        m = jnp.max(sw)
        iv = _iv()
        idx = jnp.min(jnp.where(sw == m, iv, jnp.int32(1 << 30)))
        r = jax.lax.shift_right_logical(idx, 7)
        c = jax.lax.bitwise_and(idx, jnp.int32(127))
        sel_row = li == c
        sc = jnp.sum(jnp.where(sel_row, sc_ref[pl.ds(r, 1), :], 0.0))
        st = jnp.sum(jnp.where(sel_row, st_ref[pl.ds(r, 1), :], 0.0))
        en = jnp.sum(jnp.where(sel_row, en_ref[pl.ds(r, 1), :], 0.0))
        starts = st_ref[...]
        ends = en_ref[...]
        inter = jnp.maximum(0.0, jnp.minimum(en, ends) - jnp.maximum(st, starts))
        union = (en - st) + (ends - starts) - inter
        iou = inter / jnp.maximum(union, 1e-8)
        sw = jnp.where((iou > _NMS_THR) | (iv == idx), _NEG, sw)
        row = jnp.where(li == 0, sc, jnp.where(li == 1, st,
                        jnp.where(li == 2, en, 0.0)))
        first_row = jnp.where(t == 0, row, first_row)
        out_ref[pl.ds(t, 1), :] = jnp.where(m > -1.0, row, first_row)
        return (sw, first_row)

    jax.lax.fori_loop(0, _POST_TOPN, body,
                      (swork, jnp.zeros((1, 128), jnp.float32)), unroll=2)


def _head_pallas(sequence, W_bb, b_bb, W_dw, W_pw, b_pw, bn_gamma, bn_beta,
                 bn_mean, bn_var, W_cls, b_cls, W_box, b_box):
    f32 = jnp.float32
    seq = sequence.reshape(_SEQ_LEN)
    xpad = jnp.pad(seq, (3, 4))
    taps = [jax.lax.slice(xpad, (t,), (t + 8 * (_L - 1) + 1,), (8,))
            for t in range(7)]
    T = jnp.stack(taps + [jnp.zeros((_L,), f32)], axis=1)
    W1 = jnp.concatenate([W_bb[:, 0, :].T, jnp.zeros((1, _OUT_CH), f32)], 0)
    wd = jnp.concatenate([W_dw[:, 0, :].T, jnp.zeros((5, _OUT_CH), f32)], 0)
    Wpw = W_pw[:, :, 0].T
    Wcls = W_cls[:, :, 0].T
    Wb0 = W_box[0::2, :, 0].T
    Wb1 = W_box[1::2, :, 0].T
    lens = jnp.asarray(_ANCHOR_LENGTHS, f32).reshape(1, _A)
    args = (T, W1, b_bb.reshape(1, -1), wd, Wpw, b_pw.reshape(1, -1),
            bn_gamma.reshape(1, -1), bn_beta.reshape(1, -1),
            bn_mean.reshape(1, -1), bn_var.reshape(1, -1),
            Wcls, b_cls.reshape(1, -1), Wb0, b_box[0::2].reshape(1, -1),
            Wb1, b_box[1::2].reshape(1, -1), lens)
    return pl.pallas_call(
        _head_kernel,
        out_shape=[jax.ShapeDtypeStruct((_L, _A), f32)] * 3,
    )(*args)


def _nms_pallas(s96, st96, en96):
    out = pl.pallas_call(
        _nms_kernel,
        out_shape=jax.ShapeDtypeStruct((_POST_TOPN + 4, 128), jnp.float32),
    )(s96, st96, en96)
    return out[:_POST_TOPN, :3][:, None, :]


def kernel(sequence, W_bb, b_bb, W_dw, W_pw, b_pw, bn_gamma, bn_beta,
           bn_mean, bn_var, W_cls, b_cls, W_box, b_box):
    sc, st, en = _head_pallas(sequence, W_bb, b_bb, W_dw, W_pw, b_pw,
                              bn_gamma, bn_beta, bn_mean, bn_var,
                              W_cls, b_cls, W_box, b_box)
    return _nms_pallas(sc.reshape(_ROWS, 128), st.reshape(_ROWS, 128),
                       en.reshape(_ROWS, 128))


# speedup vs baseline: 1.3312x; 1.0134x over previous
import jax
import jax.numpy as jnp
from jax.experimental import pallas as pl

_OUT_CH = 64
_RPN_CH = 16
_A = 6
_ANCHOR_LENGTHS = (16.0, 32.0, 64.0, 128.0, 256.0, 512.0)
_STRIDE = 8
_PRE_TOPN = 6000
_NMS_THR = 0.7
_POST_TOPN = 300
_SEQ_LEN = 16384
_L = _SEQ_LEN // _STRIDE
_N = _L * _A
_ROWS = _N // 128
_NEG = -1e30


def _head_kernel(T_ref, W1_ref, bbb_ref, wd_ref, Wpw_ref, bpw_ref,
                 gam_ref, bet_ref, mu_ref, var_ref,
                 Wcls_ref, bcls_ref, Wb0_ref, bb0_ref, Wb1_ref, bb1_ref,
                 lens_ref, sc_ref, st_ref, en_ref):
    bf = jnp.bfloat16
    def mdot(a, b):
        return jnp.dot(a.astype(bf), b.astype(bf),
                       preferred_element_type=jnp.float32)
    feat = mdot(T_ref[...], W1_ref[...]) + bbb_ref[0:1, :]
    feat = jnp.maximum(feat, 0.0)
    featb = feat.astype(bf).astype(jnp.float32)
    wd = wd_ref[...]
    zrow = jnp.zeros((1, _OUT_CH), jnp.float32)
    fm1 = jnp.concatenate([zrow, featb[:-1, :]], axis=0)
    fp1 = jnp.concatenate([featb[1:, :], zrow], axis=0)
    x = fm1 * wd[0:1, :] + featb * wd[1:2, :] + fp1 * wd[2:3, :]
    x = mdot(x, Wpw_ref[...]) + bpw_ref[0:1, :]
    x = jnp.maximum(x, 0.0)
    x = gam_ref[0:1, :] * (x - mu_ref[0:1, :]) / jnp.sqrt(var_ref[0:1, :] + 1e-5) \
        + bet_ref[0:1, :]
    logits = mdot(x, Wcls_ref[...]) + bcls_ref[0:1, :]
    scores = jax.nn.sigmoid(logits)
    d0 = mdot(x, Wb0_ref[...]) + bb0_ref[0:1, :]
    d1 = mdot(x, Wb1_ref[...]) + bb1_ref[0:1, :]
    anc_c = (jax.lax.broadcasted_iota(jnp.int32, (_L, _A), 0).astype(jnp.float32)
             * _STRIDE + _STRIDE / 2.0)
    anc_l = lens_ref[0:1, :]
    new_c = anc_c + d0 * anc_l
    new_l = anc_l * jnp.exp(jnp.clip(d1, -10.0, 10.0))
    sc_ref[...] = scores
    st_ref[...] = jnp.clip(new_c - new_l / 2.0, 0.0, float(_SEQ_LEN))
    en_ref[...] = jnp.clip(new_c + new_l / 2.0, 0.0, float(_SEQ_LEN))


def _iv():
    return (jax.lax.broadcasted_iota(jnp.int32, (_ROWS, 128), 0) * 128
            + jax.lax.broadcasted_iota(jnp.int32, (_ROWS, 128), 1))


def _nms_kernel(sc_ref, st_ref, en_ref, out_ref):
    scores = sc_ref[...]
    ivals = _iv()
    sbits = jax.lax.bitcast_convert_type(scores, jnp.int32)

    def t_quad(_, lh):
        lo, hi = lh
        q = jax.lax.div(hi - lo, jnp.int32(4))
        m1, m2, m3 = lo + q, lo + 2 * q, lo + 3 * q
        c1 = jnp.sum(jnp.where(sbits >= m1, 1.0, 0.0)) >= _PRE_TOPN
        c2 = jnp.sum(jnp.where(sbits >= m2, 1.0, 0.0)) >= _PRE_TOPN
        c3 = jnp.sum(jnp.where(sbits >= m3, 1.0, 0.0)) >= _PRE_TOPN
        lo = jnp.where(c3, m3, jnp.where(c2, m2, jnp.where(c1, m1, lo)))
        hi = jnp.where(c3, hi, jnp.where(c2, m3, jnp.where(c1, m2, m1)))
        return (lo, hi)

    def t_body(_, lh):
        lo, hi = lh
        mid = jax.lax.div(lo + hi, jnp.int32(2))
        ge = jnp.sum(jnp.where(sbits >= mid, 1.0, 0.0)) >= _PRE_TOPN
        return (jnp.where(ge, mid, lo), jnp.where(ge, hi, mid))

    lh = jax.lax.fori_loop(0, 14, t_quad, (jnp.int32(0), jnp.int32(1 << 30)))
    tstar, _ = jax.lax.fori_loop(0, 3, t_body, lh)
    cnt_gt = jnp.sum(jnp.where(sbits > tstar, 1.0, 0.0))
    need = jnp.float32(_PRE_TOPN) - cnt_gt
    tie = sbits == tstar

    def c_quad(_, lh):
        lo, hi = lh
        q = jax.lax.div(hi - lo, jnp.int32(4))
        m1, m2, m3 = lo + q, lo + 2 * q, lo + 3 * q
        c1 = jnp.sum(jnp.where(tie & (ivals <= m1), 1.0, 0.0)) >= need
        c2 = jnp.sum(jnp.where(tie & (ivals <= m2), 1.0, 0.0)) >= need
        c3 = jnp.sum(jnp.where(tie & (ivals <= m3), 1.0, 0.0)) >= need
        lo2 = jnp.where(c1, lo, jnp.where(c2, m1, jnp.where(c3, m2, m3)))
        hi2 = jnp.where(c1, m1, jnp.where(c2, m2, jnp.where(c3, m3, hi)))
        return (lo2, hi2)

    def c_body(_, lh):
        lo, hi = lh
        mid = lo + jax.lax.div(hi - lo, jnp.int32(2))
        ge = jnp.sum(jnp.where(tie & (ivals <= mid), 1.0, 0.0)) >= need
        return (jnp.where(ge, lo, mid), jnp.where(ge, mid, hi))

    lh = jax.lax.fori_loop(0, 7, c_quad, (jnp.int32(-1), jnp.int32(_N - 1)))
    _, cstar = jax.lax.fori_loop(0, 3, c_body, lh)
    eligible = (sbits > tstar) | (tie & (ivals <= cstar))
    swork = jnp.where(eligible, scores, _NEG)

    li = jax.lax.broadcasted_iota(jnp.int32, (1, 128), 1)

    def body(t, carry):
        sw, first_row = carry
        m = jnp.max(sw)
        iv = _iv()
        idx = jnp.min(jnp.where(sw == m, iv, jnp.int32(1 << 30)))
        r = jax.lax.shift_right_logical(idx, 7)
        c = jax.lax.bitwise_and(idx, jnp.int32(127))
        sel_row = li == c
        sc = jnp.sum(jnp.where(sel_row, sc_ref[pl.ds(r, 1), :], 0.0))
        st = jnp.sum(jnp.where(sel_row, st_ref[pl.ds(r, 1), :], 0.0))
        en = jnp.sum(jnp.where(sel_row, en_ref[pl.ds(r, 1), :], 0.0))
        starts = st_ref[...]
        ends = en_ref[...]
        inter = jnp.maximum(0.0, jnp.minimum(en, ends) - jnp.maximum(st, starts))
        union = (en - st) + (ends - starts) - inter
        iou = inter / jnp.maximum(union, 1e-8)
        sw = jnp.where((iou > _NMS_THR) | (iv == idx), _NEG, sw)
        row = jnp.where(li == 0, sc, jnp.where(li == 1, st,
                        jnp.where(li == 2, en, 0.0)))
        first_row = jnp.where(t == 0, row, first_row)
        out_ref[pl.ds(t, 1), :] = jnp.where(m > -1.0, row, first_row)
        return (sw, first_row)

    jax.lax.fori_loop(0, _POST_TOPN, body,
                      (swork, jnp.zeros((1, 128), jnp.float32)), unroll=2)


def _head_pallas(sequence, W_bb, b_bb, W_dw, W_pw, b_pw, bn_gamma, bn_beta,
                 bn_mean, bn_var, W_cls, b_cls, W_box, b_box):
    f32 = jnp.float32
    seq = sequence.reshape(_SEQ_LEN)
    xpad = jnp.pad(seq, (3, 4))
    taps = [jax.lax.slice(xpad, (t,), (t + 8 * (_L - 1) + 1,), (8,))
            for t in range(7)]
    T = jnp.stack(taps + [jnp.zeros((_L,), f32)], axis=1)
    W1 = jnp.concatenate([W_bb[:, 0, :].T, jnp.zeros((1, _OUT_CH), f32)], 0)
    wd = jnp.concatenate([W_dw[:, 0, :].T, jnp.zeros((5, _OUT_CH), f32)], 0)
    Wpw = W_pw[:, :, 0].T
    Wcls = W_cls[:, :, 0].T
    Wb0 = W_box[0::2, :, 0].T
    Wb1 = W_box[1::2, :, 0].T
    lens = jnp.asarray(_ANCHOR_LENGTHS, f32).reshape(1, _A)
    args = (T, W1, b_bb.reshape(1, -1), wd, Wpw, b_pw.reshape(1, -1),
            bn_gamma.reshape(1, -1), bn_beta.reshape(1, -1),
            bn_mean.reshape(1, -1), bn_var.reshape(1, -1),
            Wcls, b_cls.reshape(1, -1), Wb0, b_box[0::2].reshape(1, -1),
            Wb1, b_box[1::2].reshape(1, -1), lens)
    return pl.pallas_call(
        _head_kernel,
        out_shape=[jax.ShapeDtypeStruct((_L, _A), f32)] * 3,
    )(*args)


def _nms_pallas(s96, st96, en96):
    out = pl.pallas_call(
        _nms_kernel,
        out_shape=jax.ShapeDtypeStruct((_POST_TOPN + 4, 128), jnp.float32),
    )(s96, st96, en96)
    return out[:_POST_TOPN, :3][:, None, :]


def kernel(sequence, W_bb, b_bb, W_dw, W_pw, b_pw, bn_gamma, bn_beta,
           bn_mean, bn_var, W_cls, b_cls, W_box, b_box):
    sc, st, en = _head_pallas(sequence, W_bb, b_bb, W_dw, W_pw, b_pw,
                              bn_gamma, bn_beta, bn_mean, bn_var,
                              W_cls, b_cls, W_box, b_box)
    return _nms_pallas(sc.reshape(_ROWS, 128), st.reshape(_ROWS, 128),
                       en.reshape(_ROWS, 128))
